# baseline (speedup 1.0000x reference)
import jax
import jax.numpy as jnp
from jax import lax
from jax.experimental import pallas as pl
from jax.experimental.pallas import tpu as pltpu, tpu_sc as plsc


_B, _S, _F, _D = 2, 2048, 64, 256
_MAX_CLASSES = 10
_S_CHUNK = 128
_NSB = _S // _S_CHUNK

_NC, _NS, _L = 2, 16, 16
_NW = _NC * _NS
_NB = (_B * _S) // _NW


def _label_gather(lab_hbm, msk_hbm, ltab_hbm, out_hbm, lab_v, msk_v, idx_v,
                  rows_v, sem):
    wid = lax.axis_index("s") * _NC + lax.axis_index("c")
    base = wid * _NB
    pltpu.sync_copy(lab_hbm.at[pl.ds(base, _NB)], lab_v)
    pltpu.sync_copy(msk_hbm.at[pl.ds(base, _NB)], msk_v)
    for i in range(_NB // _L):
        lab16 = lab_v[pl.ds(i * _L, _L)]
        m16 = msk_v[pl.ds(i * _L, _L)]
        idx_v[pl.ds(i * _L, _L)] = lab16 * m16 + _MAX_CLASSES * (1 - m16)
    pltpu.async_copy(ltab_hbm.at[idx_v], rows_v, sem).wait()
    pltpu.sync_copy(rows_v, out_hbm.at[pl.ds(base, _NB)])


def _encoder_block(feats_ref, mask_ref, labrow_ref, w_ref, bias_ref,
                   feat_tab_ref, train_tab_ref, pos_ref, out_ref):
    m = mask_ref[0, 0, :]

    t0 = train_tab_ref[0, :]
    t1 = train_tab_ref[1, :]
    m_f = m.astype(jnp.float32)[:, None]
    train_emb = t0[None, :] + m_f * (t1 - t0)[None, :]

    row = pos_ref[...] + labrow_ref[...] + train_emb

    base_f = bias_ref[...] + feat_tab_ref[...]

    feats = feats_ref[0]
    w = w_ref[0, :]
    full = (feats[:, :, None] * w[None, None, :]
            + base_f[None, :, :] + row[:, None, :])
    out_ref[...] = full.reshape(1, _S_CHUNK * _F, _D)


@jax.jit
def kernel(features, labels, is_train_mask, W_feat, b_feat, feat_idx_table,
           label_table, is_train_table, pos_table):
    b, s, f = features.shape
    d = W_feat.shape[1]
    labels = labels.astype(jnp.int32)
    is_train_mask = is_train_mask.astype(jnp.int32)

    lab_rows = pl.kernel(
        _label_gather,
        out_type=jax.ShapeDtypeStruct((_B * _S, _D), jnp.float32),
        mesh=plsc.VectorSubcoreMesh(core_axis_name="c", subcore_axis_name="s"),
        scratch_types=[
            pltpu.VMEM((_NB,), jnp.int32),
            pltpu.VMEM((_NB,), jnp.int32),
            pltpu.VMEM((_NB,), jnp.int32),
            pltpu.VMEM((_NB, _D), jnp.float32),
            pltpu.SemaphoreType.DMA,
        ],
    )(labels.reshape(_B * _S), is_train_mask.reshape(_B * _S), label_table)

    grid = (_B, _NSB)
    out = pl.pallas_call(
        _encoder_block,
        grid=grid,
        in_specs=[
            pl.BlockSpec((1, _S_CHUNK, _F), lambda b, sb: (b, sb, 0)),
            pl.BlockSpec((1, 1, _S_CHUNK), lambda b, sb: (b * _NSB + sb, 0, 0)),
            pl.BlockSpec((_S_CHUNK, _D), lambda b, sb: (b * _NSB + sb, 0)),
            pl.BlockSpec((1, _D), lambda b, sb: (0, 0)),
            pl.BlockSpec((1, _D), lambda b, sb: (0, 0)),
            pl.BlockSpec((_F, _D), lambda b, sb: (0, 0)),
            pl.BlockSpec((2, _D), lambda b, sb: (0, 0)),
            pl.BlockSpec((_S_CHUNK, _D), lambda b, sb: (sb, 0)),
        ],
        out_specs=pl.BlockSpec((1, _S_CHUNK * _F, _D), lambda b, sb: (b, sb, 0)),
        out_shape=jax.ShapeDtypeStruct((b, s * f, d), jnp.float32),
    )(features, is_train_mask.reshape(_B * _NSB, 1, _S_CHUNK), lab_rows,
      W_feat, b_feat.reshape(1, d), feat_idx_table, is_train_table, pos_table)
    return out


# speedup vs baseline: 1.6344x; 1.6344x over previous
import jax
import jax.numpy as jnp
from jax import lax
from jax.experimental import pallas as pl
from jax.experimental.pallas import tpu as pltpu, tpu_sc as plsc


_B, _S, _F, _D = 2, 2048, 64, 256
_MAX_CLASSES = 10
_S_CHUNK = 128
_NSB = _S // _S_CHUNK

_NC, _NS, _L = 2, 16, 16
_NW = _NC * _NS
_NB = (_B * _S) // _NW


def _label_gather(lab_hbm, msk_hbm, ltab_hbm, out_hbm, lab_v, msk_v,
                  tab_v, rows_v, sem):
    wid = lax.axis_index("s") * _NC + lax.axis_index("c")
    base = wid * _NB
    pltpu.sync_copy(lab_hbm.at[pl.ds(base, _NB)], lab_v)
    pltpu.sync_copy(msk_hbm.at[pl.ds(base, _NB)], msk_v)
    pltpu.sync_copy(ltab_hbm, tab_v)

    def body(g, carry):
        lab16 = lab_v[pl.ds(g * _L, _L)]
        m16 = msk_v[pl.ds(g * _L, _L)]
        idx16 = lab16 * m16 + _MAX_CLASSES * (1 - m16)
        for j in range(_L):
            idx_s = idx16[j]
            tok = g * _L + j
            for k in range(_D // _L):
                rows_v[tok, pl.ds(k * _L, _L)] = tab_v[idx_s, pl.ds(k * _L, _L)]
        return carry

    lax.fori_loop(0, _NB // _L, body, 0)
    pltpu.sync_copy(rows_v, out_hbm.at[pl.ds(base, _NB)])


def _encoder_block(feats_ref, mask_ref, labrow_ref, w_ref, bias_ref,
                   feat_tab_ref, train_tab_ref, pos_ref, out_ref):
    m = mask_ref[0, 0, :]

    t0 = train_tab_ref[0, :]
    t1 = train_tab_ref[1, :]
    m_f = m.astype(jnp.float32)[:, None]
    train_emb = t0[None, :] + m_f * (t1 - t0)[None, :]

    row = pos_ref[...] + labrow_ref[...] + train_emb

    base_f = bias_ref[...] + feat_tab_ref[...]

    feats = feats_ref[0]
    w = w_ref[0, :]
    full = (feats[:, :, None] * w[None, None, :]
            + base_f[None, :, :] + row[:, None, :])
    out_ref[...] = full.reshape(1, _S_CHUNK * _F, _D)


@jax.jit
def kernel(features, labels, is_train_mask, W_feat, b_feat, feat_idx_table,
           label_table, is_train_table, pos_table):
    b, s, f = features.shape
    d = W_feat.shape[1]
    labels = labels.astype(jnp.int32)
    is_train_mask = is_train_mask.astype(jnp.int32)

    lab_rows = pl.kernel(
        _label_gather,
        out_type=jax.ShapeDtypeStruct((_B * _S, _D), jnp.float32),
        mesh=plsc.VectorSubcoreMesh(core_axis_name="c", subcore_axis_name="s"),
        scratch_types=[
            pltpu.VMEM((_NB,), jnp.int32),
            pltpu.VMEM((_NB,), jnp.int32),
            pltpu.VMEM((_MAX_CLASSES + 1, _D), jnp.float32),
            pltpu.VMEM((_NB, _D), jnp.float32),
            pltpu.SemaphoreType.DMA,
        ],
    )(labels.reshape(_B * _S), is_train_mask.reshape(_B * _S), label_table)

    grid = (_B, _NSB)
    out = pl.pallas_call(
        _encoder_block,
        grid=grid,
        in_specs=[
            pl.BlockSpec((1, _S_CHUNK, _F), lambda b, sb: (b, sb, 0)),
            pl.BlockSpec((1, 1, _S_CHUNK), lambda b, sb: (b * _NSB + sb, 0, 0)),
            pl.BlockSpec((_S_CHUNK, _D), lambda b, sb: (b * _NSB + sb, 0)),
            pl.BlockSpec((1, _D), lambda b, sb: (0, 0)),
            pl.BlockSpec((1, _D), lambda b, sb: (0, 0)),
            pl.BlockSpec((_F, _D), lambda b, sb: (0, 0)),
            pl.BlockSpec((2, _D), lambda b, sb: (0, 0)),
            pl.BlockSpec((_S_CHUNK, _D), lambda b, sb: (sb, 0)),
        ],
        out_specs=pl.BlockSpec((1, _S_CHUNK * _F, _D), lambda b, sb: (b, sb, 0)),
        out_shape=jax.ShapeDtypeStruct((b, s * f, d), jnp.float32),
    )(features, is_train_mask.reshape(_B * _NSB, 1, _S_CHUNK), lab_rows,
      W_feat, b_feat.reshape(1, d), feat_idx_table, is_train_table, pos_table)
    return out


# speedup vs baseline: 1.7128x; 1.0480x over previous
import jax
import jax.numpy as jnp
from jax import lax
from jax.experimental import pallas as pl
from jax.experimental.pallas import tpu as pltpu, tpu_sc as plsc


_B, _S, _F, _D = 2, 2048, 64, 256
_MAX_CLASSES = 10
_S_CHUNK = 128
_NSB = _S // _S_CHUNK

_NC, _NS, _L = 2, 16, 16
_NW = _NC * _NS
_NB = _S // _NW


def _label_gather(lab_hbm, msk_hbm, ltab_hbm, out_hbm, lab_v, msk_v,
                  tab_v, rows_v):
    wid = lax.axis_index("s") * _NC + lax.axis_index("c")
    base = wid * _NB
    pltpu.sync_copy(lab_hbm.at[pl.ds(base, _NB)], lab_v)
    pltpu.sync_copy(msk_hbm.at[pl.ds(base, _NB)], msk_v)
    pltpu.sync_copy(ltab_hbm, tab_v)

    def body(g, carry):
        lab16 = lab_v[pl.ds(g * _L, _L)]
        m16 = msk_v[pl.ds(g * _L, _L)]
        idx16 = lab16 * m16 + _MAX_CLASSES * (1 - m16)
        for j in range(_L):
            idx_s = idx16[j]
            tok = g * _L + j
            for k in range(_D // _L):
                rows_v[tok, pl.ds(k * _L, _L)] = tab_v[idx_s, pl.ds(k * _L, _L)]
        return carry

    lax.fori_loop(0, _NB // _L, body, 0)
    pltpu.sync_copy(rows_v, out_hbm.at[pl.ds(base, _NB)])


def _encoder_block_onehot(feats_ref, labels_ref, mask_ref, w_ref, bias_ref,
                          feat_tab_ref, label_tab_ref, train_tab_ref, pos_ref,
                          out_ref):
    lab = labels_ref[0, 0, :]
    m = mask_ref[0, 0, :]
    lab_eff = lab * m + _MAX_CLASSES * (1 - m)

    classes = jax.lax.broadcasted_iota(jnp.int32, (_S_CHUNK, _MAX_CLASSES + 1), 1)
    onehot = (lab_eff[:, None] == classes).astype(jnp.float32)
    lab_emb = jnp.dot(onehot, label_tab_ref[...],
                      preferred_element_type=jnp.float32)

    t0 = train_tab_ref[0, :]
    t1 = train_tab_ref[1, :]
    m_f = m.astype(jnp.float32)[:, None]
    train_emb = t0[None, :] + m_f * (t1 - t0)[None, :]

    row = pos_ref[...] + lab_emb + train_emb
    base_f = bias_ref[...] + feat_tab_ref[...]

    feats = feats_ref[0]
    w = w_ref[0, :]
    full = (feats[:, :, None] * w[None, None, :]
            + base_f[None, :, :] + row[:, None, :])
    out_ref[...] = full.reshape(1, _S_CHUNK * _F, _D)


def _encoder_block_rows(feats_ref, mask_ref, labrow_ref, w_ref, bias_ref,
                        feat_tab_ref, train_tab_ref, pos_ref, alias_ref,
                        out_ref):
    del alias_ref
    m = mask_ref[0, 0, :]
    t0 = train_tab_ref[0, :]
    t1 = train_tab_ref[1, :]
    m_f = m.astype(jnp.float32)[:, None]
    train_emb = t0[None, :] + m_f * (t1 - t0)[None, :]

    row = pos_ref[...] + labrow_ref[...] + train_emb
    base_f = bias_ref[...] + feat_tab_ref[...]

    feats = feats_ref[0]
    w = w_ref[0, :]
    full = (feats[:, :, None] * w[None, None, :]
            + base_f[None, :, :] + row[:, None, :])
    out_ref[...] = full.reshape(1, _S_CHUNK * _F, _D)


@jax.jit
def kernel(features, labels, is_train_mask, W_feat, b_feat, feat_idx_table,
           label_table, is_train_table, pos_table):
    b, s, f = features.shape
    d = W_feat.shape[1]
    labels = labels.astype(jnp.int32)
    is_train_mask = is_train_mask.astype(jnp.int32)
    labels3 = labels.reshape(_B * _NSB, 1, _S_CHUNK)
    mask3 = is_train_mask.reshape(_B * _NSB, 1, _S_CHUNK)
    bias2 = b_feat.reshape(1, d)

    lab_rows = pl.kernel(
        _label_gather,
        out_type=jax.ShapeDtypeStruct((_S, _D), jnp.float32),
        mesh=plsc.VectorSubcoreMesh(core_axis_name="c", subcore_axis_name="s"),
        scratch_types=[
            pltpu.VMEM((_NB,), jnp.int32),
            pltpu.VMEM((_NB,), jnp.int32),
            pltpu.VMEM((_MAX_CLASSES + 1, _D), jnp.float32),
            pltpu.VMEM((_NB, _D), jnp.float32),
        ],
    )(labels[1], is_train_mask[1], label_table)

    out_shape = jax.ShapeDtypeStruct((b, s * f, d), jnp.float32)

    out0 = pl.pallas_call(
        _encoder_block_onehot,
        grid=(_NSB,),
        in_specs=[
            pl.BlockSpec((1, _S_CHUNK, _F), lambda sb: (0, sb, 0)),
            pl.BlockSpec((1, 1, _S_CHUNK), lambda sb: (sb, 0, 0)),
            pl.BlockSpec((1, 1, _S_CHUNK), lambda sb: (sb, 0, 0)),
            pl.BlockSpec((1, _D), lambda sb: (0, 0)),
            pl.BlockSpec((1, _D), lambda sb: (0, 0)),
            pl.BlockSpec((_F, _D), lambda sb: (0, 0)),
            pl.BlockSpec((_MAX_CLASSES + 1, _D), lambda sb: (0, 0)),
            pl.BlockSpec((2, _D), lambda sb: (0, 0)),
            pl.BlockSpec((_S_CHUNK, _D), lambda sb: (sb, 0)),
        ],
        out_specs=pl.BlockSpec((1, _S_CHUNK * _F, _D), lambda sb: (0, sb, 0)),
        out_shape=out_shape,
    )(features, labels3, mask3, W_feat, bias2, feat_idx_table, label_table,
      is_train_table, pos_table)

    out = pl.pallas_call(
        _encoder_block_rows,
        grid=(_NSB,),
        in_specs=[
            pl.BlockSpec((1, _S_CHUNK, _F), lambda sb: (1, sb, 0)),
            pl.BlockSpec((1, 1, _S_CHUNK), lambda sb: (_NSB + sb, 0, 0)),
            pl.BlockSpec((_S_CHUNK, _D), lambda sb: (sb, 0)),
            pl.BlockSpec((1, _D), lambda sb: (0, 0)),
            pl.BlockSpec((1, _D), lambda sb: (0, 0)),
            pl.BlockSpec((_F, _D), lambda sb: (0, 0)),
            pl.BlockSpec((2, _D), lambda sb: (0, 0)),
            pl.BlockSpec((_S_CHUNK, _D), lambda sb: (sb, 0)),
            pl.BlockSpec(memory_space=pltpu.MemorySpace.HBM),
        ],
        out_specs=pl.BlockSpec((1, _S_CHUNK * _F, _D), lambda sb: (1, sb, 0)),
        out_shape=out_shape,
        input_output_aliases={8: 0},
    )(features, mask3, lab_rows, W_feat, bias2, feat_idx_table,
      is_train_table, pos_table, out0)
    return out
